# baseline (speedup 1.0000x reference)
import functools

import jax
import jax.numpy as jnp
from jax import lax
from jax.experimental import pallas as pl
from jax.experimental.pallas import tpu as pltpu
from jax.experimental.pallas import tpu_sc as plsc

INPUT_DIM = 512
N_THRESH = 8
NUM_FACTS = INPUT_DIM * N_THRESH
BATCH = 16384
LANES = 16
NC = 2
NS = 16
NW = NC * NS
ROWS_PER_W = BATCH // NW
RBLK = 8
NBLK = ROWS_PER_W // RBLK
NGRP = INPUT_DIM // LANES


def _sc_body(x_hbm, th_hbm, lk_hbm, out_hbm, xbuf, obuf, abuf, cbuf,
             sem_in, sem_out):
    wid = lax.axis_index("s") * NC + lax.axis_index("c")
    row0 = wid * ROWS_PER_W

    pltpu.sync_copy(th_hbm, cbuf)
    pltpu.sync_copy(lk_hbm, abuf)

    def init_body(i, _):
        lkv = abuf[pl.ds(i * LANES, LANES)]
        kv = jnp.minimum(jnp.maximum(jnp.exp(lkv), 0.5), 50.0)
        tv = cbuf[pl.ds(i * LANES, LANES)]
        abuf[pl.ds(i * LANES, LANES)] = kv
        cbuf[pl.ds(i * LANES, LANES)] = kv * tv
        return 0

    lax.fori_loop(0, NUM_FACTS // LANES, init_body, 0)

    def blk_body(g, _):
        r0 = row0 + g * RBLK
        pltpu.sync_copy(x_hbm.at[pl.ds(r0, RBLK)], xbuf)

        def grp_body(m, _):
            for r in range(RBLK):
                xv = xbuf[r, pl.ds(m * LANES, LANES)]
                for t in range(N_THRESH):
                    idx = lax.shift_right_logical(lax.iota(jnp.int32, LANES), 3) + (2 * t)
                    xg = lax.gather(
                        xv, idx[:, None],
                        lax.GatherDimensionNumbers(
                            offset_dims=(), collapsed_slice_dims=(0,),
                            start_index_map=(0,)),
                        slice_sizes=(1,),
                        mode=lax.GatherScatterMode.PROMISE_IN_BOUNDS)
                    col = m * N_THRESH * LANES + t * LANES
                    av = abuf[pl.ds(col, LANES)]
                    cv = cbuf[pl.ds(col, LANES)]
                    e = jnp.exp(cv - av * xg)
                    obuf[r, pl.ds(col, LANES)] = 1.0 / (1.0 + e)
            return 0

        lax.fori_loop(0, NGRP, grp_body, 0)
        pltpu.sync_copy(obuf, out_hbm.at[pl.ds(r0, RBLK)])
        return 0

    lax.fori_loop(0, NBLK, blk_body, 0)


def kernel(x, th, log_kappa, feat_idx):
    del feat_idx
    mesh = plsc.VectorSubcoreMesh(core_axis_name="c", subcore_axis_name="s")
    f = functools.partial(
        pl.kernel,
        mesh=mesh,
        out_type=jax.ShapeDtypeStruct((BATCH, NUM_FACTS), jnp.float32),
        scratch_types=[
            pltpu.VMEM((RBLK, INPUT_DIM), jnp.float32),
            pltpu.VMEM((RBLK, NUM_FACTS), jnp.float32),
            pltpu.VMEM((NUM_FACTS,), jnp.float32),
            pltpu.VMEM((NUM_FACTS,), jnp.float32),
            pltpu.SemaphoreType.DMA,
            pltpu.SemaphoreType.DMA,
        ],
    )(_sc_body)
    return f(x, th, log_kappa)


# speedup vs baseline: 7.3319x; 7.3319x over previous
import functools

import jax
import jax.numpy as jnp
from jax import lax
from jax.experimental import pallas as pl
from jax.experimental.pallas import tpu as pltpu
from jax.experimental.pallas import tpu_sc as plsc

INPUT_DIM = 512
N_THRESH = 8
NUM_FACTS = INPUT_DIM * N_THRESH
BATCH = 16384
BLOCK_B = 1024

LANES = 16
NC = 2
NS = 16
NW = NC * NS
SC_ROWS = 1024
ROWS_PER_W = SC_ROWS // NW
RBLK = 8
NBLK = ROWS_PER_W // RBLK
NGRP = INPUT_DIM // LANES
TC_ROWS = BATCH - SC_ROWS


def _tc_body(x_ref, g_ref, th_ref, lk_ref, out_ref):
    xb = x_ref[...].astype(jnp.bfloat16)
    xg = jax.lax.dot_general(
        xb, g_ref[...], (((1,), (0,)), ((), ())),
        preferred_element_type=jnp.float32,
    )
    a = 0.5 * jnp.clip(jnp.exp(lk_ref[...]), 0.5, 50.0)
    c = a * th_ref[...]
    out_ref[...] = 0.5 + 0.5 * jnp.tanh(a * xg - c)


def _tc_part(x, g, th2, lk2):
    grid = (TC_ROWS // BLOCK_B,)
    return pl.pallas_call(
        _tc_body,
        grid=grid,
        in_specs=[
            pl.BlockSpec((BLOCK_B, INPUT_DIM), lambda i: (i, 0)),
            pl.BlockSpec((INPUT_DIM, NUM_FACTS), lambda i: (0, 0)),
            pl.BlockSpec((1, NUM_FACTS), lambda i: (0, 0)),
            pl.BlockSpec((1, NUM_FACTS), lambda i: (0, 0)),
        ],
        out_specs=pl.BlockSpec((BLOCK_B, NUM_FACTS), lambda i: (i, 0)),
        out_shape=jax.ShapeDtypeStruct((TC_ROWS, NUM_FACTS), jnp.float32),
    )(x, g, th2, lk2)


def _sc_body(x_hbm, th_hbm, lk_hbm, out_hbm, xbuf, obuf, abuf, cbuf,
             sem_in, sem_out):
    wid = lax.axis_index("s") * NC + lax.axis_index("c")
    row0 = wid * ROWS_PER_W

    pltpu.sync_copy(th_hbm, cbuf)
    pltpu.sync_copy(lk_hbm, abuf)

    def init_body(i, _):
        lkv = abuf[pl.ds(i * LANES, LANES)]
        kv = jnp.minimum(jnp.maximum(jnp.exp(lkv), 0.5), 50.0)
        tv = cbuf[pl.ds(i * LANES, LANES)]
        abuf[pl.ds(i * LANES, LANES)] = kv
        cbuf[pl.ds(i * LANES, LANES)] = kv * tv
        return 0

    lax.fori_loop(0, NUM_FACTS // LANES, init_body, 0)

    def blk_body(g, _):
        r0 = row0 + g * RBLK
        pltpu.sync_copy(x_hbm.at[pl.ds(r0, RBLK)], xbuf)

        def grp_body(m, _):
            for r in range(RBLK):
                xv = xbuf[r, pl.ds(m * LANES, LANES)]
                for t in range(N_THRESH):
                    idx = lax.shift_right_logical(
                        lax.iota(jnp.int32, LANES), 3) + (2 * t)
                    xg = lax.gather(
                        xv, idx[:, None],
                        lax.GatherDimensionNumbers(
                            offset_dims=(), collapsed_slice_dims=(0,),
                            start_index_map=(0,)),
                        slice_sizes=(1,),
                        mode=lax.GatherScatterMode.PROMISE_IN_BOUNDS)
                    col = m * N_THRESH * LANES + t * LANES
                    av = abuf[pl.ds(col, LANES)]
                    cv = cbuf[pl.ds(col, LANES)]
                    e = jnp.exp(cv - av * xg)
                    obuf[r, pl.ds(col, LANES)] = 1.0 / (1.0 + e)
            return 0

        lax.fori_loop(0, NGRP, grp_body, 0)
        pltpu.sync_copy(obuf, out_hbm.at[pl.ds(r0, RBLK)])
        return 0

    lax.fori_loop(0, NBLK, blk_body, 0)


def _sc_part(x_tail, th, lk):
    mesh = plsc.VectorSubcoreMesh(core_axis_name="c", subcore_axis_name="s")
    f = functools.partial(
        pl.kernel,
        mesh=mesh,
        out_type=jax.ShapeDtypeStruct((SC_ROWS, NUM_FACTS), jnp.float32),
        scratch_types=[
            pltpu.VMEM((RBLK, INPUT_DIM), jnp.float32),
            pltpu.VMEM((RBLK, NUM_FACTS), jnp.float32),
            pltpu.VMEM((NUM_FACTS,), jnp.float32),
            pltpu.VMEM((NUM_FACTS,), jnp.float32),
            pltpu.SemaphoreType.DMA,
            pltpu.SemaphoreType.DMA,
        ],
    )(_sc_body)
    return f(x_tail, th, lk)


def kernel(x, th, log_kappa, feat_idx):
    g = (feat_idx[None, :] == jnp.arange(INPUT_DIM, dtype=feat_idx.dtype)[:, None])
    g = g.astype(jnp.bfloat16)
    th2 = th.reshape(1, NUM_FACTS)
    lk2 = log_kappa.reshape(1, NUM_FACTS)
    out_tc = _tc_part(x[:TC_ROWS], g, th2, lk2)
    out_sc = _sc_part(x[TC_ROWS:], th, log_kappa)
    return jnp.concatenate([out_tc, out_sc], axis=0)


# speedup vs baseline: 21.4864x; 2.9305x over previous
import jax
import jax.numpy as jnp
from jax.experimental import pallas as pl

INPUT_DIM = 512
N_THRESH = 8
NUM_FACTS = INPUT_DIM * N_THRESH
BATCH = 16384
BLOCK_B = 1024
BLOCK_F = 2048


def _body(x_ref, g_ref, th_ref, lk_ref, out_ref):
    xb = x_ref[...].astype(jnp.bfloat16)
    xg = jax.lax.dot_general(
        xb, g_ref[...], (((1,), (0,)), ((), ())),
        preferred_element_type=jnp.float32,
    )
    a = 0.5 * jnp.clip(jnp.exp(lk_ref[...]), 0.5, 50.0)
    c = a * th_ref[...]
    out_ref[...] = 0.5 + 0.5 * jnp.tanh(a * xg - c)


def kernel(x, th, log_kappa, feat_idx):
    g = (feat_idx[None, :] == jnp.arange(INPUT_DIM, dtype=feat_idx.dtype)[:, None])
    g = g.astype(jnp.bfloat16)
    th2 = th.reshape(1, NUM_FACTS)
    lk2 = log_kappa.reshape(1, NUM_FACTS)
    grid = (BATCH // BLOCK_B, NUM_FACTS // BLOCK_F)
    return pl.pallas_call(
        _body,
        grid=grid,
        in_specs=[
            pl.BlockSpec((BLOCK_B, INPUT_DIM), lambda i, j: (i, 0)),
            pl.BlockSpec((INPUT_DIM, BLOCK_F), lambda i, j: (0, j)),
            pl.BlockSpec((1, BLOCK_F), lambda i, j: (0, j)),
            pl.BlockSpec((1, BLOCK_F), lambda i, j: (0, j)),
        ],
        out_specs=pl.BlockSpec((BLOCK_B, BLOCK_F), lambda i, j: (i, j)),
        out_shape=jax.ShapeDtypeStruct((BATCH, NUM_FACTS), jnp.float32),
    )(x, g, th2, lk2)


# speedup vs baseline: 26.6744x; 1.2415x over previous
import jax
import jax.numpy as jnp
from jax.experimental import pallas as pl

INPUT_DIM = 512
N_THRESH = 8
NUM_FACTS = INPUT_DIM * N_THRESH
BATCH = 16384
BLOCK_B = 1024


def _body(x_ref, g_ref, th_ref, lk_ref, out_ref):
    xb = x_ref[...].astype(jnp.bfloat16)
    xg = jax.lax.dot_general(
        xb, g_ref[...], (((1,), (0,)), ((), ())),
        preferred_element_type=jnp.float32,
    )
    a = 0.5 * jnp.clip(jnp.exp(lk_ref[...]), 0.5, 50.0)
    c = a * th_ref[...]
    out_ref[...] = 0.5 + 0.5 * jnp.tanh(xg - c)


def kernel(x, th, log_kappa, feat_idx):
    g = (feat_idx[None, :] == jnp.arange(INPUT_DIM, dtype=feat_idx.dtype)[:, None])
    a = 0.5 * jnp.clip(jnp.exp(log_kappa), 0.5, 50.0)
    g = (g * a[None, :]).astype(jnp.bfloat16)
    th2 = th.reshape(1, NUM_FACTS)
    lk2 = log_kappa.reshape(1, NUM_FACTS)
    grid = (BATCH // BLOCK_B,)
    return pl.pallas_call(
        _body,
        grid=grid,
        in_specs=[
            pl.BlockSpec((BLOCK_B, INPUT_DIM), lambda i: (i, 0)),
            pl.BlockSpec((INPUT_DIM, NUM_FACTS), lambda i: (0, 0)),
            pl.BlockSpec((1, NUM_FACTS), lambda i: (0, 0)),
            pl.BlockSpec((1, NUM_FACTS), lambda i: (0, 0)),
        ],
        out_specs=pl.BlockSpec((BLOCK_B, NUM_FACTS), lambda i: (i, 0)),
        out_shape=jax.ShapeDtypeStruct((BATCH, NUM_FACTS), jnp.float32),
    )(x, g, th2, lk2)
